# baseline (speedup 1.0000x reference)
import functools

import jax
import jax.numpy as jnp
from jax import lax
from jax.experimental import pallas as pl
from jax.experimental.pallas import tpu as pltpu
from jax.experimental.pallas import tpu_sc as plsc

N = 10000
D = 256
K = 16
NPAD = 10240
QB = 256
G = 80
L = 128
BIGI = 2 ** 30

NW = 32
PER_W = NPAD // NW
CH = 8
NCH = PER_W // CH

RB = 1024


def _knn_body(pt_ref, q_ref, nbr_ref, d_ref):
    q = q_ref[...]
    d = jnp.zeros((QB, G, L), jnp.float32)
    for c in range(3):
        diff = q[:, c][:, None, None] - pt_ref[c][None, :, :]
        d = d + diff * diff
    d_ref[...] = d
    gm0 = jnp.min(d, axis=1)

    sub_iota = lax.broadcasted_iota(jnp.int32, (QB, G, L), 1)
    lane_iota = lax.broadcasted_iota(jnp.int32, (QB, L), 1)
    inf = jnp.float32(jnp.inf)

    tv0 = jnp.full((K, QB), inf, jnp.float32)
    ti0 = jnp.full((K, QB), BIGI, jnp.int32)

    def cond(c):
        r, done, _, _, _ = c
        return jnp.logical_and(jnp.logical_not(done), r < K)

    def body(c):
        r, _, gm, tv, ti = c
        d = d_ref[...]
        ga = jnp.min(jnp.where(d == gm[:, None, :], sub_iota, G), axis=1)
        dnew = jnp.where(sub_iota == ga[:, None, :], inf, d)
        d_ref[...] = dnew
        gm_next = jnp.min(dnew, axis=1)
        gi = ga * L + lane_iota

        cv = jnp.concatenate([tv, gm.T], axis=0)
        ci = jnp.concatenate([ti, gi.T], axis=0)
        nv, ni = [], []
        for _ in range(K):
            m = jnp.min(cv, axis=0)
            sel = jnp.min(jnp.where(cv == m[None, :], ci, BIGI), axis=0)
            nv.append(m[None, :])
            ni.append(sel[None, :])
            kill = jnp.logical_and(cv == m[None, :], ci == sel[None, :])
            cv = jnp.where(kill, inf, cv)
        tv2 = jnp.concatenate(nv, axis=0)
        ti2 = jnp.concatenate(ni, axis=0)
        rm = jnp.min(gm_next, axis=1)
        done2 = jnp.all(rm > tv2[K - 1])
        return (r + 1, done2, gm_next, tv2, ti2)

    _, _, _, _, ti = lax.while_loop(cond, body, (0, False, gm0, tv0, ti0))
    nbr_ref[...] = ti


def _knn(pt, pos_pad):
    return pl.pallas_call(
        _knn_body,
        grid=(NPAD // QB,),
        in_specs=[
            pl.BlockSpec((3, G, L), lambda i: (0, 0, 0)),
            pl.BlockSpec((QB, 3), lambda i: (i, 0)),
        ],
        out_specs=pl.BlockSpec((K, QB), lambda i: (0, i)),
        out_shape=jax.ShapeDtypeStruct((K, NPAD), jnp.int32),
        scratch_shapes=[pltpu.VMEM((QB, G, L), jnp.float32)],
    )(pt, pos_pad)


def _gather_sum_body(x_hbm, nbrc_hbm, out_hbm, idx_all, buf0, buf1, o0, o1,
                     sem_g, sem_o):
    wid = lax.axis_index("s") * 2 + lax.axis_index("c")
    pltpu.sync_copy(nbrc_hbm.at[wid], idx_all)
    pltpu.async_copy(x_hbm.at[idx_all.at[0]], buf0, sem_g)

    def step(g, carry):
        for b, (bufb, ob) in enumerate(((buf0, o0), (buf1, o1))):
            c = 2 * g + b
            other = buf1 if b == 0 else buf0
            pltpu.make_async_copy(x_hbm.at[idx_all.at[c]], bufb, sem_g).wait()

            @pl.when(c + 1 < NCH)
            def _():
                pltpu.async_copy(
                    x_hbm.at[idx_all.at[c + 1]], other, sem_g)

            @pl.when(c >= 2)
            def _():
                pltpu.make_async_copy(
                    ob, out_hbm.at[pl.ds(0, CH)], sem_o).wait()

            @plsc.parallel_loop(0, CH, unroll=2)
            def _(i):
                for j in range(D // 16):
                    vs = [bufb[k * CH + i, pl.ds(16 * j, 16)]
                          for k in range(K)]
                    while len(vs) > 1:
                        vs = [vs[a] + vs[a + 1]
                              for a in range(0, len(vs), 2)]
                    ob[i, pl.ds(16 * j, 16)] = vs[0]
            nb = (wid * NCH + c) * CH
            pltpu.async_copy(ob, out_hbm.at[pl.ds(nb, CH)], sem_o)
        return carry

    lax.fori_loop(0, NCH // 2, step, 0)
    pltpu.make_async_copy(o0, out_hbm.at[pl.ds(0, CH)], sem_o).wait()
    pltpu.make_async_copy(o1, out_hbm.at[pl.ds(0, CH)], sem_o).wait()


def _gather_sum(x_pad, nbr_c):
    mesh = plsc.VectorSubcoreMesh(core_axis_name="c", subcore_axis_name="s")
    f = pl.kernel(
        _gather_sum_body,
        out_type=jax.ShapeDtypeStruct((NPAD, D), jnp.float32),
        mesh=mesh,
        scratch_types=[
            pltpu.VMEM((NCH, K * CH), jnp.int32),
            pltpu.VMEM((K * CH, D), jnp.float32),
            pltpu.VMEM((K * CH, D), jnp.float32),
            pltpu.VMEM((CH, D), jnp.float32),
            pltpu.VMEM((CH, D), jnp.float32),
            pltpu.SemaphoreType.DMA,
            pltpu.SemaphoreType.DMA,
        ],
    )
    return f(x_pad, nbr_c)


def _combine_body(x_ref, a_ref, wr_ref, wl_ref, b_ref, o_ref):
    o_ref[...] = jnp.maximum(
        jnp.dot(a_ref[...], wl_ref[...], preferred_element_type=jnp.float32)
        + jnp.dot(x_ref[...], wr_ref[...], preferred_element_type=jnp.float32)
        + b_ref[...],
        0.0,
    )


def _combine(x, agg, wr_t, wl_t, b):
    return pl.pallas_call(
        _combine_body,
        grid=(NPAD // RB,),
        in_specs=[
            pl.BlockSpec((RB, D), lambda i: (i, 0)),
            pl.BlockSpec((RB, D), lambda i: (i, 0)),
            pl.BlockSpec((D, D), lambda i: (0, 0)),
            pl.BlockSpec((D, D), lambda i: (0, 0)),
            pl.BlockSpec((1, D), lambda i: (0, 0)),
        ],
        out_specs=pl.BlockSpec((RB, D), lambda i: (i, 0)),
        out_shape=jax.ShapeDtypeStruct((NPAD, D), jnp.float32),
    )(x, agg, wr_t, wl_t, b)


def kernel(h_obs, pos_obs, W_l0, b_l0, W_r0, W_l1, b_l1, W_r1):
    pos_pad = jnp.full((NPAD, 3), 1e20, jnp.float32).at[:N].set(pos_obs)
    pt = pos_pad.T.reshape(3, G, L)
    h_pad = jnp.zeros((NPAD, D), jnp.float32).at[:N].set(h_obs)

    nbr_t = _knn(pt, pos_pad)
    nbr_c = nbr_t.reshape(K, NW, NCH, CH).transpose(1, 2, 0, 3).reshape(
        NW, NCH, K * CH)

    wl0 = (W_l0 * (1.0 / K)).T
    wr0 = W_r0.T
    wl1 = (W_l1 * (1.0 / K)).T
    wr1 = W_r1.T
    b0 = b_l0.reshape(1, D)
    b1 = b_l1.reshape(1, D)

    agg0 = _gather_sum(h_pad, nbr_c)
    h1 = _combine(h_pad, agg0, wr0, wl0, b0)
    agg1 = _gather_sum(h1, nbr_c)
    h2 = _combine(h1, agg1, wr1, wl1, b1)
    return h2[:N]


# speedup vs baseline: 1.1097x; 1.1097x over previous
import functools

import jax
import jax.numpy as jnp
from jax import lax
from jax.experimental import pallas as pl
from jax.experimental.pallas import tpu as pltpu
from jax.experimental.pallas import tpu_sc as plsc

N = 10000
D = 256
K = 16
NPAD = 10240
QB = 256
G = 80
L = 128
BIGI = 2 ** 30

NW = 32
PER_W = NPAD // NW
CH = 8
NCH = PER_W // CH

RB = 1024


def _knn_body(pt_ref, q_ref, nbr_ref, d_ref):
    q = q_ref[...]
    d = jnp.zeros((QB, G, L), jnp.float32)
    for c in range(3):
        diff = q[:, c][:, None, None] - pt_ref[c][None, :, :]
        d = d + diff * diff
    d_ref[...] = d
    gm0 = jnp.min(d, axis=1)

    sub_iota = lax.broadcasted_iota(jnp.int32, (QB, G, L), 1)
    lane_iota = lax.broadcasted_iota(jnp.int32, (QB, L), 1)
    inf = jnp.float32(jnp.inf)

    tv0 = jnp.full((K, QB), inf, jnp.float32)
    ti0 = jnp.full((K, QB), BIGI, jnp.int32)

    def cond(c):
        r, done, _, _, _ = c
        return jnp.logical_and(jnp.logical_not(done), r < K)

    def body(c):
        r, _, gm, tv, ti = c
        d = d_ref[...]
        ga = jnp.min(jnp.where(d == gm[:, None, :], sub_iota, G), axis=1)
        dnew = jnp.where(sub_iota == ga[:, None, :], inf, d)
        d_ref[...] = dnew
        gm_next = jnp.min(dnew, axis=1)
        gi = ga * L + lane_iota

        cv = jnp.concatenate([tv, gm.T], axis=0)
        ci = jnp.concatenate([ti, gi.T], axis=0)
        nv, ni = [], []
        for _ in range(K):
            m = jnp.min(cv, axis=0)
            sel = jnp.min(jnp.where(cv == m[None, :], ci, BIGI), axis=0)
            nv.append(m[None, :])
            ni.append(sel[None, :])
            kill = jnp.logical_and(cv == m[None, :], ci == sel[None, :])
            cv = jnp.where(kill, inf, cv)
        tv2 = jnp.concatenate(nv, axis=0)
        ti2 = jnp.concatenate(ni, axis=0)
        rm = jnp.min(gm_next, axis=1)
        done2 = jnp.all(rm > tv2[K - 1])
        return (r + 1, done2, gm_next, tv2, ti2)

    _, _, _, _, ti = lax.while_loop(cond, body, (0, False, gm0, tv0, ti0))
    nbr_ref[...] = ti


def _knn(pt, pos_pad):
    return pl.pallas_call(
        _knn_body,
        grid=(NPAD // QB,),
        in_specs=[
            pl.BlockSpec((3, G, L), lambda i: (0, 0, 0)),
            pl.BlockSpec((QB, 3), lambda i: (i, 0)),
        ],
        out_specs=pl.BlockSpec((K, QB), lambda i: (0, i)),
        out_shape=jax.ShapeDtypeStruct((K, NPAD), jnp.int32),
        scratch_shapes=[pltpu.VMEM((QB, G, L), jnp.float32)],
    )(pt, pos_pad)


def _gather_sum_body(x_hbm, nbrc_hbm, out_hbm, idx_all, buf0, buf1, o0, o1,
                     sem_g, sem_o):
    wid = lax.axis_index("s") * 2 + lax.axis_index("c")
    pltpu.sync_copy(nbrc_hbm.at[wid], idx_all)
    pltpu.async_copy(x_hbm.at[idx_all.at[0]], buf0, sem_g)

    def step(g, carry):
        for b, (bufb, ob) in enumerate(((buf0, o0), (buf1, o1))):
            c = 2 * g + b
            other = buf1 if b == 0 else buf0
            pltpu.make_async_copy(x_hbm.at[idx_all.at[c]], bufb, sem_g).wait()

            @pl.when(c + 1 < NCH)
            def _():
                pltpu.async_copy(
                    x_hbm.at[idx_all.at[c + 1]], other, sem_g)

            @pl.when(c >= 2)
            def _():
                pltpu.make_async_copy(
                    ob, out_hbm.at[pl.ds(0, CH)], sem_o).wait()

            def acc_node(i, carry2):
                for j in range(D // 16):
                    vs = [bufb[k * CH + i, pl.ds(16 * j, 16)]
                          for k in range(K)]
                    while len(vs) > 1:
                        vs = [vs[a] + vs[a + 1]
                              for a in range(0, len(vs), 2)]
                    ob[i, pl.ds(16 * j, 16)] = vs[0]
                return carry2

            lax.fori_loop(0, CH, acc_node, 0)
            nb = (wid * NCH + c) * CH
            pltpu.async_copy(ob, out_hbm.at[pl.ds(nb, CH)], sem_o)
        return carry

    lax.fori_loop(0, NCH // 2, step, 0)
    pltpu.make_async_copy(o0, out_hbm.at[pl.ds(0, CH)], sem_o).wait()
    pltpu.make_async_copy(o1, out_hbm.at[pl.ds(0, CH)], sem_o).wait()


def _gather_sum(x_pad, nbr_c):
    mesh = plsc.VectorSubcoreMesh(core_axis_name="c", subcore_axis_name="s")
    f = pl.kernel(
        _gather_sum_body,
        out_type=jax.ShapeDtypeStruct((NPAD, D), jnp.float32),
        mesh=mesh,
        scratch_types=[
            pltpu.VMEM((NCH, K * CH), jnp.int32),
            pltpu.VMEM((K * CH, D), jnp.float32),
            pltpu.VMEM((K * CH, D), jnp.float32),
            pltpu.VMEM((CH, D), jnp.float32),
            pltpu.VMEM((CH, D), jnp.float32),
            pltpu.SemaphoreType.DMA,
            pltpu.SemaphoreType.DMA,
        ],
    )
    return f(x_pad, nbr_c)


def _combine_body(x_ref, a_ref, wr_ref, wl_ref, b_ref, o_ref):
    o_ref[...] = jnp.maximum(
        jnp.dot(a_ref[...], wl_ref[...], preferred_element_type=jnp.float32)
        + jnp.dot(x_ref[...], wr_ref[...], preferred_element_type=jnp.float32)
        + b_ref[...],
        0.0,
    )


def _combine(x, agg, wr_t, wl_t, b):
    return pl.pallas_call(
        _combine_body,
        grid=(NPAD // RB,),
        in_specs=[
            pl.BlockSpec((RB, D), lambda i: (i, 0)),
            pl.BlockSpec((RB, D), lambda i: (i, 0)),
            pl.BlockSpec((D, D), lambda i: (0, 0)),
            pl.BlockSpec((D, D), lambda i: (0, 0)),
            pl.BlockSpec((1, D), lambda i: (0, 0)),
        ],
        out_specs=pl.BlockSpec((RB, D), lambda i: (i, 0)),
        out_shape=jax.ShapeDtypeStruct((NPAD, D), jnp.float32),
    )(x, agg, wr_t, wl_t, b)


def kernel(h_obs, pos_obs, W_l0, b_l0, W_r0, W_l1, b_l1, W_r1):
    pos_pad = jnp.full((NPAD, 3), 1e20, jnp.float32).at[:N].set(pos_obs)
    pt = pos_pad.T.reshape(3, G, L)
    h_pad = jnp.zeros((NPAD, D), jnp.float32).at[:N].set(h_obs)

    nbr_t = _knn(pt, pos_pad)
    nbr_c = nbr_t.reshape(K, NW, NCH, CH).transpose(1, 2, 0, 3).reshape(
        NW, NCH, K * CH)

    wl0 = (W_l0 * (1.0 / K)).T
    wr0 = W_r0.T
    wl1 = (W_l1 * (1.0 / K)).T
    wr1 = W_r1.T
    b0 = b_l0.reshape(1, D)
    b1 = b_l1.reshape(1, D)

    agg0 = _gather_sum(h_pad, nbr_c)
    h1 = _combine(h_pad, agg0, wr0, wl0, b0)
    agg1 = _gather_sum(h1, nbr_c)
    h2 = _combine(h1, agg1, wr1, wl1, b1)
    return h2[:N]
